# Initial kernel scaffold; baseline (speedup 1.0000x reference)
import jax
import jax.numpy as jnp
from jax.experimental import pallas as pl


def kernel(samples, encoded_pixels, counts):
    raise NotImplementedError("write your pallas kernel here")


# speedup vs baseline: 57.8891x; 57.8891x over previous
import functools

import jax
import jax.numpy as jnp
from jax import lax
from jax.experimental import pallas as pl
from jax.experimental.pallas import tpu as pltpu
from jax.experimental.pallas import tpu_sc as plsc

NC, NS = 2, 16
NW = NC * NS
LANES = 128
WIN = 1024
ROWS = WIN // LANES


def _build_kernels(B, C, H, W, P):
    HW = H * W
    N = B * HW
    M = N // NW
    NWIN = M // WIN
    TPF = HW // M
    P_pad = -(-P // (NS * WIN)) * (NS * WIN)
    ZB = P_pad // NS
    NV = ZB // WIN
    IROWS = M // LANES

    mesh = plsc.VectorSubcoreMesh(core_axis_name="c", subcore_axis_name="s")

    def tables(t0, t1, t2):
        return ((0, t0), (1, t1), (2, t2))

    @functools.partial(
        pl.kernel,
        out_type=jax.ShapeDtypeStruct((NC * C * P_pad,), jnp.float32),
        mesh=mesh,
        scratch_types=[
            pltpu.VMEM_SHARED((P_pad,), jnp.float32),
            pltpu.VMEM_SHARED((P_pad,), jnp.float32),
            pltpu.VMEM_SHARED((P_pad,), jnp.float32),
            pltpu.VMEM((WIN,), jnp.float32),
            pltpu.VMEM((ROWS, LANES), jnp.int32),
            pltpu.VMEM((C, ROWS, LANES), jnp.float32),
        ],
    )
    def scatter_kernel(idx_hbm, samp_hbm, part_hbm, t0, t1, t2, zbuf,
                       idx_buf, upd_buf):
        core = lax.axis_index("c")
        sub = lax.axis_index("s")
        wid = core * NS + sub

        @pl.loop(0, WIN // 16)
        def _zf(i):
            zbuf[pl.ds(i * 16, 16)] = jnp.zeros((16,), jnp.float32)

        for _, t in tables(t0, t1, t2):
            @pl.loop(0, NV)
            def _zero(v, t=t):
                pltpu.sync_copy(zbuf, t.at[pl.ds(sub * ZB + v * WIN, WIN)])

        plsc.subcore_barrier()

        irow0 = wid * IROWS
        b = wid // TPF
        hw0 = (wid % TPF) * M

        @pl.loop(0, NWIN)
        def _scan(w):
            irow = pl.multiple_of(irow0 + w * ROWS, 8)
            pltpu.sync_copy(idx_hbm.at[pl.ds(irow, ROWS)], idx_buf)
            for c in range(C):
                srow = ((b * C + c) * HW + hw0) // LANES + w * ROWS
                srow = pl.multiple_of(srow, 8)
                pltpu.sync_copy(samp_hbm.at[pl.ds(srow, ROWS)],
                                upd_buf.at[c])
            for c, t in tables(t0, t1, t2):
                for j in range(ROWS):
                    pltpu.sync_copy(upd_buf.at[c, j],
                                    t.at[idx_buf.at[j]], add=True)

        plsc.subcore_barrier()

        for c, t in tables(t0, t1, t2):
            @pl.loop(0, NV)
            def _dump(v, c=c, t=t):
                off = sub * ZB + v * WIN
                poff = pl.multiple_of((core * C + c) * P_pad + off, 8)
                pltpu.sync_copy(t.at[pl.ds(off, WIN)],
                                part_hbm.at[pl.ds(poff, WIN)])

    @functools.partial(
        pl.kernel,
        out_type=jax.ShapeDtypeStruct((N * C // LANES, LANES), jnp.float32),
        mesh=mesh,
        scratch_types=[
            pltpu.VMEM_SHARED((P_pad,), jnp.float32),
            pltpu.VMEM_SHARED((P_pad,), jnp.float32),
            pltpu.VMEM_SHARED((P_pad,), jnp.float32),
            pltpu.VMEM((WIN,), jnp.float32),
            pltpu.VMEM((WIN,), jnp.float32),
            pltpu.VMEM((WIN,), jnp.float32),
            pltpu.VMEM((ROWS, LANES), jnp.int32),
            pltpu.VMEM((ROWS, LANES), jnp.float32),
        ],
    )
    def gather_kernel(part_hbm, cnt_hbm, idx_hbm, out_hbm, t0, t1, t2,
                      pa, pb, cb, idx_buf, gbuf):
        core = lax.axis_index("c")
        sub = lax.axis_index("s")
        wid = core * NS + sub

        for c, t in tables(t0, t1, t2):
            @pl.loop(0, NV)
            def _merge(v, c=c, t=t):
                off = sub * ZB + v * WIN
                p0 = pl.multiple_of(c * P_pad + off, 8)
                p1 = pl.multiple_of((C + c) * P_pad + off, 8)
                pltpu.sync_copy(part_hbm.at[pl.ds(p0, WIN)], pa)
                pltpu.sync_copy(part_hbm.at[pl.ds(p1, WIN)], pb)
                pltpu.sync_copy(cnt_hbm.at[pl.ds(off, WIN)], cb)

                @pl.loop(0, WIN // 16)
                def _div(k):
                    s = pl.ds(k * 16, 16)
                    pa[s] = (pa[s] + pb[s]) / cb[s]

                pltpu.sync_copy(pa, t.at[pl.ds(off, WIN)])

        plsc.subcore_barrier()

        irow0 = wid * IROWS
        b = wid // TPF
        hw0 = (wid % TPF) * M

        @pl.loop(0, NWIN)
        def _gather(w):
            irow = pl.multiple_of(irow0 + w * ROWS, 8)
            pltpu.sync_copy(idx_hbm.at[pl.ds(irow, ROWS)], idx_buf)
            for c, t in tables(t0, t1, t2):
                for j in range(ROWS):
                    pltpu.sync_copy(t.at[idx_buf.at[j]], gbuf.at[j])
                orow = ((b * C + c) * HW + hw0) // LANES + w * ROWS
                orow = pl.multiple_of(orow, 8)
                pltpu.sync_copy(gbuf, out_hbm.at[pl.ds(orow, ROWS)])

    return scatter_kernel, gather_kernel, P_pad


def kernel(samples, encoded_pixels, counts):
    B, C, H, W = samples.shape
    P = counts.shape[0]
    N = B * H * W

    scatter_kernel, gather_kernel, P_pad = _build_kernels(B, C, H, W, P)

    idx2d = encoded_pixels.reshape(N // LANES, LANES)
    samp2d = samples.reshape(N * C // LANES, LANES)
    cntf = counts.astype(jnp.float32).reshape(P)
    cntf = jnp.concatenate(
        [cntf, jnp.ones((P_pad - P,), jnp.float32)])

    partials = scatter_kernel(idx2d, samp2d)
    outflat = gather_kernel(partials, cntf, idx2d)
    return outflat.reshape(B, C, H, W)


# speedup vs baseline: 78.2756x; 1.3522x over previous
import functools

import jax
import jax.numpy as jnp
from jax import lax
from jax.experimental import pallas as pl
from jax.experimental.pallas import tpu as pltpu
from jax.experimental.pallas import tpu_sc as plsc

NC, NS = 2, 16
NW = NC * NS
WIN = 1024


def _build_kernels(B, C, H, W, P):
    HW = H * W
    N = B * HW
    M = N // NW
    NWIN = M // WIN
    TPF = HW // M
    P_pad = -(-P // (NS * WIN)) * (NS * WIN)
    ZB = P_pad // NS
    NV = ZB // WIN

    mesh = plsc.VectorSubcoreMesh(core_axis_name="c", subcore_axis_name="s")

    def tables(t0, t1, t2):
        return ((0, t0), (1, t1), (2, t2))

    @functools.partial(
        pl.kernel,
        out_type=jax.ShapeDtypeStruct((NC * C * P_pad,), jnp.float32),
        mesh=mesh,
        scratch_types=[
            pltpu.VMEM_SHARED((P_pad,), jnp.float32),
            pltpu.VMEM_SHARED((P_pad,), jnp.float32),
            pltpu.VMEM_SHARED((P_pad,), jnp.float32),
            pltpu.VMEM((WIN,), jnp.float32),
            pltpu.VMEM((WIN,), jnp.int32),
            pltpu.VMEM((WIN,), jnp.float32),
            pltpu.VMEM((WIN,), jnp.float32),
            pltpu.VMEM((WIN,), jnp.float32),
        ],
    )
    def scatter_kernel(idx_hbm, samp_hbm, part_hbm, t0, t1, t2, zbuf,
                       idx_buf, u0, u1, u2):
        core = lax.axis_index("c")
        sub = lax.axis_index("s")
        wid = core * NS + sub

        @pl.loop(0, WIN // 16)
        def _zf(i):
            zbuf[pl.ds(i * 16, 16)] = jnp.zeros((16,), jnp.float32)

        for _, t in tables(t0, t1, t2):
            @pl.loop(0, NV)
            def _zero(v, t=t):
                pltpu.sync_copy(zbuf, t.at[pl.ds(sub * ZB + v * WIN, WIN)])

        plsc.subcore_barrier()

        q0 = wid * M
        b = wid // TPF
        hw0 = (wid % TPF) * M

        @pl.loop(0, NWIN)
        def _scan(w):
            ioff = pl.multiple_of(q0 + w * WIN, 8)
            pltpu.sync_copy(idx_hbm.at[pl.ds(ioff, WIN)], idx_buf)
            for c, u in ((0, u0), (1, u1), (2, u2)):
                soff = pl.multiple_of((b * C + c) * HW + hw0 + w * WIN, 8)
                pltpu.sync_copy(samp_hbm.at[pl.ds(soff, WIN)], u)
            for (_, t), u in zip(tables(t0, t1, t2), (u0, u1, u2)):
                pltpu.sync_copy(u, t.at[idx_buf], add=True)

        plsc.subcore_barrier()

        for c, t in tables(t0, t1, t2):
            @pl.loop(0, NV)
            def _dump(v, c=c, t=t):
                off = sub * ZB + v * WIN
                poff = pl.multiple_of((core * C + c) * P_pad + off, 8)
                pltpu.sync_copy(t.at[pl.ds(off, WIN)],
                                part_hbm.at[pl.ds(poff, WIN)])

    @functools.partial(
        pl.kernel,
        out_type=jax.ShapeDtypeStruct((N * C,), jnp.float32),
        mesh=mesh,
        scratch_types=[
            pltpu.VMEM_SHARED((P_pad,), jnp.float32),
            pltpu.VMEM_SHARED((P_pad,), jnp.float32),
            pltpu.VMEM_SHARED((P_pad,), jnp.float32),
            pltpu.VMEM((WIN,), jnp.float32),
            pltpu.VMEM((WIN,), jnp.float32),
            pltpu.VMEM((WIN,), jnp.float32),
            pltpu.VMEM((WIN,), jnp.int32),
            pltpu.VMEM((WIN,), jnp.float32),
            pltpu.VMEM((WIN,), jnp.float32),
            pltpu.VMEM((WIN,), jnp.float32),
        ],
    )
    def gather_kernel(part_hbm, cnt_hbm, idx_hbm, out_hbm, t0, t1, t2,
                      pa, pb, cb, idx_buf, g0, g1, g2):
        core = lax.axis_index("c")
        sub = lax.axis_index("s")
        wid = core * NS + sub

        for c, t in tables(t0, t1, t2):
            @pl.loop(0, NV)
            def _merge(v, c=c, t=t):
                off = sub * ZB + v * WIN
                p0 = pl.multiple_of(c * P_pad + off, 8)
                p1 = pl.multiple_of((C + c) * P_pad + off, 8)
                pltpu.sync_copy(part_hbm.at[pl.ds(p0, WIN)], pa)
                pltpu.sync_copy(part_hbm.at[pl.ds(p1, WIN)], pb)
                pltpu.sync_copy(cnt_hbm.at[pl.ds(off, WIN)], cb)

                @pl.loop(0, WIN // 16)
                def _div(k):
                    s = pl.ds(k * 16, 16)
                    pa[s] = (pa[s] + pb[s]) / cb[s]

                pltpu.sync_copy(pa, t.at[pl.ds(off, WIN)])

        plsc.subcore_barrier()

        q0 = wid * M
        b = wid // TPF
        hw0 = (wid % TPF) * M

        @pl.loop(0, NWIN)
        def _gather(w):
            ioff = pl.multiple_of(q0 + w * WIN, 8)
            pltpu.sync_copy(idx_hbm.at[pl.ds(ioff, WIN)], idx_buf)
            for (_, t), g in zip(tables(t0, t1, t2), (g0, g1, g2)):
                pltpu.sync_copy(t.at[idx_buf], g)
            for c, g in ((0, g0), (1, g1), (2, g2)):
                ooff = pl.multiple_of((b * C + c) * HW + hw0 + w * WIN, 8)
                pltpu.sync_copy(g, out_hbm.at[pl.ds(ooff, WIN)])

    return scatter_kernel, gather_kernel, P_pad


def kernel(samples, encoded_pixels, counts):
    B, C, H, W = samples.shape
    P = counts.shape[0]
    N = B * H * W

    scatter_kernel, gather_kernel, P_pad = _build_kernels(B, C, H, W, P)

    idx1d = encoded_pixels.reshape(N)
    samp1d = samples.reshape(N * C)
    cntf = counts.astype(jnp.float32).reshape(P)
    cntf = jnp.concatenate(
        [cntf, jnp.ones((P_pad - P,), jnp.float32)])

    partials = scatter_kernel(idx1d, samp1d)
    outflat = gather_kernel(partials, cntf, idx1d)
    return outflat.reshape(B, C, H, W)


# speedup vs baseline: 135.5117x; 1.7312x over previous
import functools

import jax
import jax.numpy as jnp
from jax import lax
from jax.experimental import pallas as pl
from jax.experimental.pallas import tpu as pltpu
from jax.experimental.pallas import tpu_sc as plsc

NC, NS = 2, 16
NW = NC * NS
WIN = 4096


def _build_kernels(B, C, H, W, P):
    HW = H * W
    N = B * HW
    M = N // NW
    NWIN = M // WIN
    TPF = HW // M
    P_pad = -(-P // (NS * WIN)) * (NS * WIN)
    ZB = P_pad // NS
    NV = ZB // WIN

    mesh = plsc.VectorSubcoreMesh(core_axis_name="c", subcore_axis_name="s")

    def tables(t0, t1, t2):
        return ((0, t0), (1, t1), (2, t2))

    @functools.partial(
        pl.kernel,
        out_type=jax.ShapeDtypeStruct((NC * C * P_pad,), jnp.float32),
        mesh=mesh,
        scratch_types=[
            pltpu.VMEM_SHARED((P_pad,), jnp.float32),
            pltpu.VMEM_SHARED((P_pad,), jnp.float32),
            pltpu.VMEM_SHARED((P_pad,), jnp.float32),
            pltpu.VMEM((WIN,), jnp.float32),
            pltpu.VMEM((WIN,), jnp.int32),
            pltpu.VMEM((WIN,), jnp.float32),
            pltpu.VMEM((WIN,), jnp.float32),
            pltpu.VMEM((WIN,), jnp.float32),
        ],
    )
    def scatter_kernel(idx_hbm, samp_hbm, part_hbm, t0, t1, t2, zbuf,
                       idx_buf, u0, u1, u2):
        core = lax.axis_index("c")
        sub = lax.axis_index("s")
        wid = core * NS + sub

        @pl.loop(0, WIN // 16)
        def _zf(i):
            zbuf[pl.ds(i * 16, 16)] = jnp.zeros((16,), jnp.float32)

        for _, t in tables(t0, t1, t2):
            @pl.loop(0, NV)
            def _zero(v, t=t):
                pltpu.sync_copy(zbuf, t.at[pl.ds(sub * ZB + v * WIN, WIN)])

        plsc.subcore_barrier()

        q0 = wid * M
        b = wid // TPF
        hw0 = (wid % TPF) * M

        @pl.loop(0, NWIN)
        def _scan(w):
            ioff = pl.multiple_of(q0 + w * WIN, 8)
            pltpu.sync_copy(idx_hbm.at[pl.ds(ioff, WIN)], idx_buf)
            for c, u in ((0, u0), (1, u1), (2, u2)):
                soff = pl.multiple_of((b * C + c) * HW + hw0 + w * WIN, 8)
                pltpu.sync_copy(samp_hbm.at[pl.ds(soff, WIN)], u)
            for (_, t), u in zip(tables(t0, t1, t2), (u0, u1, u2)):
                pltpu.sync_copy(u, t.at[idx_buf], add=True)

        plsc.subcore_barrier()

        for c, t in tables(t0, t1, t2):
            @pl.loop(0, NV)
            def _dump(v, c=c, t=t):
                off = sub * ZB + v * WIN
                poff = pl.multiple_of((core * C + c) * P_pad + off, 8)
                pltpu.sync_copy(t.at[pl.ds(off, WIN)],
                                part_hbm.at[pl.ds(poff, WIN)])

    @functools.partial(
        pl.kernel,
        out_type=jax.ShapeDtypeStruct((N * C,), jnp.float32),
        mesh=mesh,
        scratch_types=[
            pltpu.VMEM_SHARED((P_pad,), jnp.float32),
            pltpu.VMEM_SHARED((P_pad,), jnp.float32),
            pltpu.VMEM_SHARED((P_pad,), jnp.float32),
            pltpu.VMEM((WIN,), jnp.float32),
            pltpu.VMEM((WIN,), jnp.float32),
            pltpu.VMEM((WIN,), jnp.float32),
            pltpu.VMEM((WIN,), jnp.int32),
            pltpu.VMEM((WIN,), jnp.float32),
            pltpu.VMEM((WIN,), jnp.float32),
            pltpu.VMEM((WIN,), jnp.float32),
        ],
    )
    def gather_kernel(part_hbm, cnt_hbm, idx_hbm, out_hbm, t0, t1, t2,
                      pa, pb, cb, idx_buf, g0, g1, g2):
        core = lax.axis_index("c")
        sub = lax.axis_index("s")
        wid = core * NS + sub

        for c, t in tables(t0, t1, t2):
            @pl.loop(0, NV)
            def _merge(v, c=c, t=t):
                off = sub * ZB + v * WIN
                p0 = pl.multiple_of(c * P_pad + off, 8)
                p1 = pl.multiple_of((C + c) * P_pad + off, 8)
                pltpu.sync_copy(part_hbm.at[pl.ds(p0, WIN)], pa)
                pltpu.sync_copy(part_hbm.at[pl.ds(p1, WIN)], pb)
                pltpu.sync_copy(cnt_hbm.at[pl.ds(off, WIN)], cb)

                @pl.loop(0, WIN // 16)
                def _div(k):
                    s = pl.ds(k * 16, 16)
                    pa[s] = (pa[s] + pb[s]) / cb[s]

                pltpu.sync_copy(pa, t.at[pl.ds(off, WIN)])

        plsc.subcore_barrier()

        q0 = wid * M
        b = wid // TPF
        hw0 = (wid % TPF) * M

        @pl.loop(0, NWIN)
        def _gather(w):
            ioff = pl.multiple_of(q0 + w * WIN, 8)
            pltpu.sync_copy(idx_hbm.at[pl.ds(ioff, WIN)], idx_buf)
            for (_, t), g in zip(tables(t0, t1, t2), (g0, g1, g2)):
                pltpu.sync_copy(t.at[idx_buf], g)
            for c, g in ((0, g0), (1, g1), (2, g2)):
                ooff = pl.multiple_of((b * C + c) * HW + hw0 + w * WIN, 8)
                pltpu.sync_copy(g, out_hbm.at[pl.ds(ooff, WIN)])

    return scatter_kernel, gather_kernel, P_pad


def kernel(samples, encoded_pixels, counts):
    B, C, H, W = samples.shape
    P = counts.shape[0]
    N = B * H * W

    scatter_kernel, gather_kernel, P_pad = _build_kernels(B, C, H, W, P)

    idx1d = encoded_pixels.reshape(N)
    samp1d = samples.reshape(N * C)
    cntf = counts.astype(jnp.float32).reshape(P)
    cntf = jnp.concatenate(
        [cntf, jnp.ones((P_pad - P,), jnp.float32)])

    partials = scatter_kernel(idx1d, samp1d)
    outflat = gather_kernel(partials, cntf, idx1d)
    return outflat.reshape(B, C, H, W)


# speedup vs baseline: 163.1613x; 1.2040x over previous
import functools

import jax
import jax.numpy as jnp
from jax import lax
from jax.experimental import pallas as pl
from jax.experimental.pallas import tpu as pltpu
from jax.experimental.pallas import tpu_sc as plsc

NC, NS = 2, 16
NW = NC * NS
WIN = 4096


def _build_kernels(B, C, H, W, P):
    HW = H * W
    N = B * HW
    M = N // NW
    NWIN = M // WIN
    TPF = HW // M
    P_pad = -(-P // (NS * WIN)) * (NS * WIN)
    ZB = P_pad // NS
    NV = ZB // WIN

    mesh = plsc.VectorSubcoreMesh(core_axis_name="c", subcore_axis_name="s")

    def run_batch(copies):
        descs = [pltpu.make_async_copy(s, d, sem) for s, d, sem, _ in copies]
        for desc, (_, _, _, add) in zip(descs, copies):
            desc.start(add=add)
        for desc in descs:
            desc.wait()

    @functools.partial(
        pl.kernel,
        out_type=jax.ShapeDtypeStruct((NC * C * P_pad,), jnp.float32),
        mesh=mesh,
        scratch_types=[
            pltpu.VMEM_SHARED((P_pad,), jnp.float32),
            pltpu.VMEM_SHARED((P_pad,), jnp.float32),
            pltpu.VMEM_SHARED((P_pad,), jnp.float32),
            pltpu.VMEM((WIN,), jnp.float32),
            pltpu.VMEM((WIN,), jnp.int32),
            pltpu.VMEM((WIN,), jnp.float32),
            pltpu.VMEM((WIN,), jnp.float32),
            pltpu.VMEM((WIN,), jnp.float32),
            pltpu.SemaphoreType.DMA,
            pltpu.SemaphoreType.DMA,
        ],
    )
    def scatter_kernel(idx_hbm, samp_hbm, part_hbm, t0, t1, t2, zbuf,
                       idx_buf, u0, u1, u2, in_sem, st_sem):
        core = lax.axis_index("c")
        sub = lax.axis_index("s")
        wid = core * NS + sub
        ts = (t0, t1, t2)
        us = (u0, u1, u2)

        @pl.loop(0, WIN // 16)
        def _zf(i):
            zbuf[pl.ds(i * 16, 16)] = jnp.zeros((16,), jnp.float32)

        @pl.loop(0, NV)
        def _zero(v):
            for t in ts:
                pltpu.sync_copy(zbuf, t.at[pl.ds(sub * ZB + v * WIN, WIN)])

        plsc.subcore_barrier()

        q0 = wid * M
        b = wid // TPF
        hw0 = (wid % TPF) * M

        @pl.loop(0, NWIN)
        def _scan(w):
            ioff = pl.multiple_of(q0 + w * WIN, 8)
            ins = [(idx_hbm.at[pl.ds(ioff, WIN)], idx_buf, in_sem, False)]
            for c, u in enumerate(us):
                soff = pl.multiple_of((b * C + c) * HW + hw0 + w * WIN, 8)
                ins.append((samp_hbm.at[pl.ds(soff, WIN)], u, in_sem,
                            False))
            run_batch(ins)
            run_batch([(u, t.at[idx_buf], st_sem, True)
                       for u, t in zip(us, ts)])

        plsc.subcore_barrier()

        dumps = []
        for c, t in enumerate(ts):
            off = sub * ZB
            poff = pl.multiple_of((core * C + c) * P_pad + off, 8)
            dumps.append((t.at[pl.ds(off, ZB)],
                          part_hbm.at[pl.ds(poff, ZB)], in_sem, False))
        run_batch(dumps)

    @functools.partial(
        pl.kernel,
        out_type=jax.ShapeDtypeStruct((N * C,), jnp.float32),
        mesh=mesh,
        scratch_types=[
            pltpu.VMEM_SHARED((P_pad,), jnp.float32),
            pltpu.VMEM_SHARED((P_pad,), jnp.float32),
            pltpu.VMEM_SHARED((P_pad,), jnp.float32),
            pltpu.VMEM((WIN,), jnp.float32),
            pltpu.VMEM((WIN,), jnp.float32),
            pltpu.VMEM((WIN,), jnp.float32),
            pltpu.VMEM((WIN,), jnp.int32),
            pltpu.VMEM((WIN,), jnp.float32),
            pltpu.VMEM((WIN,), jnp.float32),
            pltpu.VMEM((WIN,), jnp.float32),
            pltpu.SemaphoreType.DMA,
            pltpu.SemaphoreType.DMA,
        ],
    )
    def gather_kernel(part_hbm, cnt_hbm, idx_hbm, out_hbm, t0, t1, t2,
                      pa, pb, cb, idx_buf, g0, g1, g2, in_sem, g_sem):
        core = lax.axis_index("c")
        sub = lax.axis_index("s")
        wid = core * NS + sub
        ts = (t0, t1, t2)
        gbufs = (g0, g1, g2)

        for c, t in enumerate(ts):
            @pl.loop(0, NV)
            def _merge(v, c=c, t=t):
                off = sub * ZB + v * WIN
                p0 = pl.multiple_of(c * P_pad + off, 8)
                p1 = pl.multiple_of((C + c) * P_pad + off, 8)
                run_batch([
                    (part_hbm.at[pl.ds(p0, WIN)], pa, in_sem, False),
                    (part_hbm.at[pl.ds(p1, WIN)], pb, in_sem, False),
                    (cnt_hbm.at[pl.ds(off, WIN)], cb, in_sem, False),
                ])

                @pl.loop(0, WIN // 16)
                def _div(k):
                    s = pl.ds(k * 16, 16)
                    pa[s] = (pa[s] + pb[s]) / cb[s]

                pltpu.sync_copy(pa, t.at[pl.ds(off, WIN)])

        plsc.subcore_barrier()

        q0 = wid * M
        b = wid // TPF
        hw0 = (wid % TPF) * M

        @pl.loop(0, NWIN)
        def _gather(w):
            ioff = pl.multiple_of(q0 + w * WIN, 8)
            pltpu.sync_copy(idx_hbm.at[pl.ds(ioff, WIN)], idx_buf)
            run_batch([(t.at[idx_buf], g, g_sem, False)
                       for t, g in zip(ts, gbufs)])
            outs = []
            for c, g in enumerate(gbufs):
                ooff = pl.multiple_of((b * C + c) * HW + hw0 + w * WIN, 8)
                outs.append((g, out_hbm.at[pl.ds(ooff, WIN)], in_sem,
                             False))
            run_batch(outs)

    return scatter_kernel, gather_kernel, P_pad


def kernel(samples, encoded_pixels, counts):
    B, C, H, W = samples.shape
    P = counts.shape[0]
    N = B * H * W

    scatter_kernel, gather_kernel, P_pad = _build_kernels(B, C, H, W, P)

    idx1d = encoded_pixels.reshape(N)
    samp1d = samples.reshape(N * C)
    cntf = counts.astype(jnp.float32).reshape(P)
    cntf = jnp.concatenate(
        [cntf, jnp.ones((P_pad - P,), jnp.float32)])

    partials = scatter_kernel(idx1d, samp1d)
    outflat = gather_kernel(partials, cntf, idx1d)
    return outflat.reshape(B, C, H, W)


# speedup vs baseline: 163.7154x; 1.0034x over previous
import functools

import jax
import jax.numpy as jnp
from jax import lax
from jax.experimental import pallas as pl
from jax.experimental.pallas import tpu as pltpu
from jax.experimental.pallas import tpu_sc as plsc

NC, NS = 2, 16
NW = NC * NS
WIN = 8192
ZWIN = 2048


def _build_kernels(B, C, H, W, P):
    HW = H * W
    N = B * HW
    M = N // NW
    NWIN = M // WIN
    TPF = HW // M
    P_pad = -(-P // (NS * ZWIN)) * (NS * ZWIN)
    ZB = P_pad // NS
    NV = ZB // ZWIN

    mesh = plsc.VectorSubcoreMesh(core_axis_name="c", subcore_axis_name="s")

    def run_batch(copies):
        descs = [pltpu.make_async_copy(s, d, sem) for s, d, sem, _ in copies]
        for desc, (_, _, _, add) in zip(descs, copies):
            desc.start(add=add)
        for desc in descs:
            desc.wait()

    @functools.partial(
        pl.kernel,
        out_type=jax.ShapeDtypeStruct((NC * C * P_pad,), jnp.float32),
        mesh=mesh,
        scratch_types=[
            pltpu.VMEM_SHARED((P_pad,), jnp.float32),
            pltpu.VMEM_SHARED((P_pad,), jnp.float32),
            pltpu.VMEM_SHARED((P_pad,), jnp.float32),
            pltpu.VMEM((ZWIN,), jnp.float32),
            pltpu.VMEM((WIN,), jnp.int32),
            pltpu.VMEM((WIN,), jnp.float32),
            pltpu.VMEM((WIN,), jnp.float32),
            pltpu.VMEM((WIN,), jnp.float32),
            pltpu.SemaphoreType.DMA,
            pltpu.SemaphoreType.DMA,
        ],
    )
    def scatter_kernel(idx_hbm, samp_hbm, part_hbm, t0, t1, t2, zbuf,
                       idx_buf, u0, u1, u2, in_sem, st_sem):
        core = lax.axis_index("c")
        sub = lax.axis_index("s")
        wid = core * NS + sub
        ts = (t0, t1, t2)
        us = (u0, u1, u2)

        @pl.loop(0, ZWIN // 16)
        def _zf(i):
            zbuf[pl.ds(i * 16, 16)] = jnp.zeros((16,), jnp.float32)

        @pl.loop(0, NV)
        def _zero(v):
            for t in ts:
                pltpu.sync_copy(zbuf, t.at[pl.ds(sub * ZB + v * ZWIN, ZWIN)])

        plsc.subcore_barrier()

        q0 = wid * M
        b = wid // TPF
        hw0 = (wid % TPF) * M

        @pl.loop(0, NWIN)
        def _scan(w):
            ioff = pl.multiple_of(q0 + w * WIN, 8)
            ins = [(idx_hbm.at[pl.ds(ioff, WIN)], idx_buf, in_sem, False)]
            for c, u in enumerate(us):
                soff = pl.multiple_of((b * C + c) * HW + hw0 + w * WIN, 8)
                ins.append((samp_hbm.at[pl.ds(soff, WIN)], u, in_sem,
                            False))
            run_batch(ins)
            run_batch([(u, t.at[idx_buf], st_sem, True)
                       for u, t in zip(us, ts)])

        plsc.subcore_barrier()

        dumps = []
        for c, t in enumerate(ts):
            off = sub * ZB
            poff = pl.multiple_of((core * C + c) * P_pad + off, 8)
            dumps.append((t.at[pl.ds(off, ZB)],
                          part_hbm.at[pl.ds(poff, ZB)], in_sem, False))
        run_batch(dumps)

    @functools.partial(
        pl.kernel,
        out_type=jax.ShapeDtypeStruct((N * C,), jnp.float32),
        mesh=mesh,
        scratch_types=[
            pltpu.VMEM_SHARED((P_pad,), jnp.float32),
            pltpu.VMEM_SHARED((P_pad,), jnp.float32),
            pltpu.VMEM_SHARED((P_pad,), jnp.float32),
            pltpu.VMEM((ZWIN,), jnp.float32),
            pltpu.VMEM((ZWIN,), jnp.float32),
            pltpu.VMEM((ZWIN,), jnp.float32),
            pltpu.VMEM((WIN,), jnp.int32),
            pltpu.VMEM((WIN,), jnp.float32),
            pltpu.VMEM((WIN,), jnp.float32),
            pltpu.VMEM((WIN,), jnp.float32),
            pltpu.SemaphoreType.DMA,
            pltpu.SemaphoreType.DMA,
        ],
    )
    def gather_kernel(part_hbm, cnt_hbm, idx_hbm, out_hbm, t0, t1, t2,
                      pa, pb, cb, idx_buf, g0, g1, g2, in_sem, g_sem):
        core = lax.axis_index("c")
        sub = lax.axis_index("s")
        wid = core * NS + sub
        ts = (t0, t1, t2)
        gbufs = (g0, g1, g2)

        for c, t in enumerate(ts):
            @pl.loop(0, NV)
            def _merge(v, c=c, t=t):
                off = sub * ZB + v * ZWIN
                p0 = pl.multiple_of(c * P_pad + off, 8)
                p1 = pl.multiple_of((C + c) * P_pad + off, 8)
                run_batch([
                    (part_hbm.at[pl.ds(p0, ZWIN)], pa, in_sem, False),
                    (part_hbm.at[pl.ds(p1, ZWIN)], pb, in_sem, False),
                    (cnt_hbm.at[pl.ds(off, ZWIN)], cb, in_sem, False),
                ])

                @pl.loop(0, ZWIN // 16)
                def _div(k):
                    s = pl.ds(k * 16, 16)
                    pa[s] = (pa[s] + pb[s]) / cb[s]

                pltpu.sync_copy(pa, t.at[pl.ds(off, ZWIN)])

        plsc.subcore_barrier()

        q0 = wid * M
        b = wid // TPF
        hw0 = (wid % TPF) * M

        @pl.loop(0, NWIN)
        def _gather(w):
            ioff = pl.multiple_of(q0 + w * WIN, 8)
            pltpu.sync_copy(idx_hbm.at[pl.ds(ioff, WIN)], idx_buf)
            run_batch([(t.at[idx_buf], g, g_sem, False)
                       for t, g in zip(ts, gbufs)])
            outs = []
            for c, g in enumerate(gbufs):
                ooff = pl.multiple_of((b * C + c) * HW + hw0 + w * WIN, 8)
                outs.append((g, out_hbm.at[pl.ds(ooff, WIN)], in_sem,
                             False))
            run_batch(outs)

    return scatter_kernel, gather_kernel, P_pad


def kernel(samples, encoded_pixels, counts):
    B, C, H, W = samples.shape
    P = counts.shape[0]
    N = B * H * W

    scatter_kernel, gather_kernel, P_pad = _build_kernels(B, C, H, W, P)

    idx1d = encoded_pixels.reshape(N)
    samp1d = samples.reshape(N * C)
    cntf = counts.astype(jnp.float32).reshape(P)
    cntf = jnp.concatenate(
        [cntf, jnp.ones((P_pad - P,), jnp.float32)])

    partials = scatter_kernel(idx1d, samp1d)
    outflat = gather_kernel(partials, cntf, idx1d)
    return outflat.reshape(B, C, H, W)


# speedup vs baseline: 165.5250x; 1.0111x over previous
import functools

import jax
import jax.numpy as jnp
from jax import lax
from jax.experimental import pallas as pl
from jax.experimental.pallas import tpu as pltpu
from jax.experimental.pallas import tpu_sc as plsc

NC, NS = 2, 16
NW = NC * NS
WIN = 8192
WIN_B = 4096
ZWIN = 4096


def _build_kernels(B, C, H, W, P):
    HW = H * W
    N = B * HW
    M = N // NW
    NWIN = M // WIN
    TPF = HW // M
    P_pad = -(-P // (NS * ZWIN)) * (NS * ZWIN)
    ZB = P_pad // NS
    NV = ZB // ZWIN

    mesh = plsc.VectorSubcoreMesh(core_axis_name="c", subcore_axis_name="s")

    def run_batch(copies):
        descs = [pltpu.make_async_copy(s, d, sem) for s, d, sem, _ in copies]
        for desc, (_, _, _, add) in zip(descs, copies):
            desc.start(add=add)
        for desc in descs:
            desc.wait()

    @functools.partial(
        pl.kernel,
        out_type=jax.ShapeDtypeStruct((NC * C * P_pad,), jnp.float32),
        mesh=mesh,
        scratch_types=[
            pltpu.VMEM_SHARED((P_pad,), jnp.float32),
            pltpu.VMEM_SHARED((P_pad,), jnp.float32),
            pltpu.VMEM_SHARED((P_pad,), jnp.float32),
            pltpu.VMEM((ZWIN,), jnp.float32),
            pltpu.VMEM((WIN,), jnp.int32),
            pltpu.VMEM((WIN,), jnp.float32),
            pltpu.VMEM((WIN,), jnp.float32),
            pltpu.VMEM((WIN,), jnp.float32),
            pltpu.SemaphoreType.DMA,
            pltpu.SemaphoreType.DMA,
        ],
    )
    def scatter_kernel(idx_hbm, samp_hbm, part_hbm, t0, t1, t2, zbuf,
                       idx_buf, u0, u1, u2, in_sem, st_sem):
        core = lax.axis_index("c")
        sub = lax.axis_index("s")
        wid = core * NS + sub
        ts = (t0, t1, t2)
        us = (u0, u1, u2)

        @pl.loop(0, ZWIN // 16)
        def _zf(i):
            zbuf[pl.ds(i * 16, 16)] = jnp.zeros((16,), jnp.float32)

        @pl.loop(0, NV)
        def _zero(v):
            for t in ts:
                pltpu.sync_copy(zbuf, t.at[pl.ds(sub * ZB + v * ZWIN, ZWIN)])

        plsc.subcore_barrier()

        q0 = wid * M
        b = wid // TPF
        hw0 = (wid % TPF) * M

        @pl.loop(0, NWIN)
        def _scan(w):
            ioff = pl.multiple_of(q0 + w * WIN, 8)
            ins = [(idx_hbm.at[pl.ds(ioff, WIN)], idx_buf, in_sem, False)]
            for c, u in enumerate(us):
                soff = pl.multiple_of((b * C + c) * HW + hw0 + w * WIN, 8)
                ins.append((samp_hbm.at[pl.ds(soff, WIN)], u, in_sem,
                            False))
            run_batch(ins)
            run_batch([(u, t.at[idx_buf], st_sem, True)
                       for u, t in zip(us, ts)])

        plsc.subcore_barrier()

        dumps = []
        for c, t in enumerate(ts):
            off = sub * ZB
            poff = pl.multiple_of((core * C + c) * P_pad + off, 8)
            dumps.append((t.at[pl.ds(off, ZB)],
                          part_hbm.at[pl.ds(poff, ZB)], in_sem, False))
        run_batch(dumps)

    @functools.partial(
        pl.kernel,
        out_type=jax.ShapeDtypeStruct((N * C,), jnp.float32),
        mesh=mesh,
        scratch_types=[
            pltpu.VMEM_SHARED((P_pad,), jnp.float32),
            pltpu.VMEM_SHARED((P_pad,), jnp.float32),
            pltpu.VMEM_SHARED((P_pad,), jnp.float32),
            pltpu.VMEM((ZWIN,), jnp.float32),
            pltpu.VMEM((ZWIN,), jnp.float32),
            pltpu.VMEM((ZWIN,), jnp.float32),
            pltpu.VMEM((WIN_B,), jnp.int32),
            pltpu.VMEM((WIN_B,), jnp.float32),
            pltpu.VMEM((WIN_B,), jnp.float32),
            pltpu.VMEM((WIN_B,), jnp.float32),
            pltpu.SemaphoreType.DMA,
            pltpu.SemaphoreType.DMA,
        ],
    )
    def gather_kernel(part_hbm, cnt_hbm, idx_hbm, out_hbm, t0, t1, t2,
                      pa, pb, cb, idx_buf, g0, g1, g2, in_sem, g_sem):
        core = lax.axis_index("c")
        sub = lax.axis_index("s")
        wid = core * NS + sub
        ts = (t0, t1, t2)
        gbufs = (g0, g1, g2)

        for c, t in enumerate(ts):
            @pl.loop(0, NV)
            def _merge(v, c=c, t=t):
                off = sub * ZB + v * ZWIN
                p0 = pl.multiple_of(c * P_pad + off, 8)
                p1 = pl.multiple_of((C + c) * P_pad + off, 8)
                run_batch([
                    (part_hbm.at[pl.ds(p0, ZWIN)], pa, in_sem, False),
                    (part_hbm.at[pl.ds(p1, ZWIN)], pb, in_sem, False),
                    (cnt_hbm.at[pl.ds(off, ZWIN)], cb, in_sem, False),
                ])

                @pl.loop(0, ZWIN // 16)
                def _div(k):
                    s = pl.ds(k * 16, 16)
                    pa[s] = (pa[s] + pb[s]) / cb[s]

                pltpu.sync_copy(pa, t.at[pl.ds(off, ZWIN)])

        plsc.subcore_barrier()

        q0 = wid * M
        b = wid // TPF
        hw0 = (wid % TPF) * M

        @pl.loop(0, M // WIN_B)
        def _gather(w):
            ioff = pl.multiple_of(q0 + w * WIN_B, 8)
            pltpu.sync_copy(idx_hbm.at[pl.ds(ioff, WIN_B)], idx_buf)
            run_batch([(t.at[idx_buf], g, g_sem, False)
                       for t, g in zip(ts, gbufs)])
            outs = []
            for c, g in enumerate(gbufs):
                ooff = pl.multiple_of((b * C + c) * HW + hw0 + w * WIN_B, 8)
                outs.append((g, out_hbm.at[pl.ds(ooff, WIN_B)], in_sem,
                             False))
            run_batch(outs)

    return scatter_kernel, gather_kernel, P_pad


def kernel(samples, encoded_pixels, counts):
    B, C, H, W = samples.shape
    P = counts.shape[0]
    N = B * H * W

    scatter_kernel, gather_kernel, P_pad = _build_kernels(B, C, H, W, P)

    idx1d = encoded_pixels.reshape(N)
    samp1d = samples.reshape(N * C)
    cntf = counts.astype(jnp.float32).reshape(P)
    cntf = jnp.concatenate(
        [cntf, jnp.ones((P_pad - P,), jnp.float32)])

    partials = scatter_kernel(idx1d, samp1d)
    outflat = gather_kernel(partials, cntf, idx1d)
    return outflat.reshape(B, C, H, W)
